# Initial kernel scaffold; baseline (speedup 1.0000x reference)
import jax
import jax.numpy as jnp
from jax.experimental import pallas as pl


def kernel(ypred, truthMask):
    raise NotImplementedError("write your pallas kernel here")


# speedup vs baseline: 1.8182x; 1.8182x over previous
import functools

import jax
import jax.numpy as jnp
from jax.experimental import pallas as pl
from jax.experimental.pallas import tpu as pltpu

TH, TW = 15, 15
PAD_HT = (TH - 1) // 2
PAD_WL = (TW - 1) // 2
MARGIN = 0.25
GAMMA = 256.0
NEG = -1e30


def _loss_body(x_ref, m_ref, o_ref, acc, *, nrows, r0, r1, c0, c1):
    i = pl.program_id(0)

    @pl.when(i == 0)
    def _init():
        acc[0] = NEG
        acc[1] = 0.0
        acc[2] = NEG
        acc[3] = 0.0

    x = x_ref[...]
    tm = m_ref[...]
    finite = jnp.isfinite(x)
    row = jax.lax.broadcasted_iota(jnp.int32, x.shape, 0) + i * nrows
    col = jax.lax.broadcasted_iota(jnp.int32, x.shape, 1)
    valid = (row >= r0) & (row < r1) & (col >= c0) & (col < c1)
    pmask = tm & finite
    nmask = valid & (~tm) & finite

    ap = jnp.maximum(1.0 + MARGIN - x, 0.0)
    an = jnp.maximum(x + MARGIN, 0.0)
    lp = jnp.where(pmask, -ap * (x - (1.0 - MARGIN)) * GAMMA, NEG)
    ln = jnp.where(nmask, an * (x - MARGIN) * GAMMA, NEG)

    mp_old = acc[0]
    mn_old = acc[2]
    mp = jnp.maximum(mp_old, jnp.max(lp))
    mn = jnp.maximum(mn_old, jnp.max(ln))
    l_sel = jnp.where(pmask, lp, ln)
    m_sel = jnp.where(pmask, mp, mn)
    e = jnp.exp(l_sel - m_sel)
    acc[0] = mp
    acc[1] = acc[1] * jnp.exp(mp_old - mp) + jnp.sum(jnp.where(pmask, e, 0.0))
    acc[2] = mn
    acc[3] = acc[3] * jnp.exp(mn_old - mn) + jnp.sum(jnp.where(nmask, e, 0.0))

    @pl.when(i == pl.num_programs(0) - 1)
    def _fin():
        z = acc[0] + jnp.log(acc[1]) + acc[2] + jnp.log(acc[3])
        o_ref[0, 0] = jnp.maximum(z, 0.0) + jnp.log1p(jnp.exp(-jnp.abs(z)))


def kernel(ypred, truthMask):
    B, H, W = ypred.shape
    mh, mw = truthMask.shape[-2], truthMask.shape[-1]
    r0 = PAD_HT - 1
    r1 = min(PAD_HT - TH + mh + 2, H)
    c0 = PAD_WL - 1
    c1 = min(PAD_WL - TW + mw + 2, W)

    x = ypred.reshape(H, W)
    tm = truthMask.reshape(H, W)

    nrows = 128 if H % 128 == 0 else H
    grid = H // nrows

    out = pl.pallas_call(
        functools.partial(
            _loss_body, nrows=nrows, r0=r0, r1=r1, c0=c0, c1=c1
        ),
        grid=(grid,),
        in_specs=[
            pl.BlockSpec((nrows, W), lambda i: (i, 0)),
            pl.BlockSpec((nrows, W), lambda i: (i, 0)),
        ],
        out_specs=pl.BlockSpec(
            (1, 1), lambda i: (0, 0), memory_space=pltpu.SMEM
        ),
        out_shape=jax.ShapeDtypeStruct((1, 1), jnp.float32),
        scratch_shapes=[pltpu.SMEM((4,), jnp.float32)],
        compiler_params=pltpu.CompilerParams(
            dimension_semantics=("arbitrary",),
        ),
    )(x, tm)
    return out.reshape(B)
